# Initial kernel scaffold; baseline (speedup 1.0000x reference)
import functools

import jax
import jax.numpy as jnp
from jax import lax
from jax.experimental import pallas as pl
from jax.experimental.pallas import tpu as pltpu
from jax.experimental.pallas import tpu_sc as plsc

NC = 2
NS = 16
NW = NC * NS
K = 80
LANES = 16


def _sc_degree(dst2d, n_nodes):
    nb = dst2d.shape[0]
    nbt = nb // NW
    rpt = n_nodes // NS
    mesh = plsc.VectorSubcoreMesh(core_axis_name="c", subcore_axis_name="s")

    @functools.partial(
        pl.kernel,
        mesh=mesh,
        out_type=jax.ShapeDtypeStruct((NC, n_nodes, LANES), jnp.float32),
        scratch_types=[
            pltpu.VMEM((nbt, K), jnp.int32),
            pltpu.VMEM((K, LANES), jnp.float32),
            pltpu.VMEM((K, LANES), jnp.float32),
            pltpu.VMEM_SHARED((n_nodes, LANES), jnp.float32),
        ],
    )
    def deg_kernel(dst_hbm, out_hbm, dstb, ones_v, zero_v, acc):
        c = lax.axis_index("c")
        s = lax.axis_index("s")
        w = c * NS + s

        @pl.loop(0, K)
        def _(r):
            ones_v[pl.ds(r, 1), :] = jnp.ones((1, LANES), jnp.float32)
            zero_v[pl.ds(r, 1), :] = jnp.zeros((1, LANES), jnp.float32)

        base_row = s * rpt
        nfull = rpt // K
        rem = rpt - nfull * K

        @pl.loop(0, nfull)
        def _(k):
            pltpu.sync_copy(zero_v, acc.at[pl.ds(base_row + k * K, K)])

        if rem:
            pltpu.sync_copy(zero_v.at[pl.ds(0, rem)],
                            acc.at[pl.ds(base_row + nfull * K, rem)])

        pltpu.sync_copy(dst_hbm.at[pl.ds(w * nbt, nbt)], dstb)
        plsc.subcore_barrier()

        @pl.loop(0, nbt)
        def _(j):
            pltpu.sync_copy(ones_v, acc.at[dstb.at[j]], add=True)

        plsc.subcore_barrier()
        pltpu.sync_copy(acc.at[pl.ds(base_row, rpt)],
                        out_hbm.at[c, pl.ds(base_row, rpt)])

    return deg_kernel(dst2d)


def _sc_segment_sum(y, src2d, dst2d):
    n_nodes, d = y.shape
    nb = src2d.shape[0]
    nbt = nb // NW
    rpt = n_nodes // NS
    mesh = plsc.VectorSubcoreMesh(core_axis_name="c", subcore_axis_name="s")

    @functools.partial(
        pl.kernel,
        mesh=mesh,
        out_type=jax.ShapeDtypeStruct((NC, n_nodes, d), jnp.float32),
        scratch_types=[
            pltpu.VMEM((nbt, K), jnp.int32),
            pltpu.VMEM((nbt, K), jnp.int32),
            pltpu.VMEM((K, d), jnp.float32),
            pltpu.VMEM((K, d), jnp.float32),
            pltpu.VMEM_SHARED((n_nodes, d), jnp.float32),
            pltpu.SemaphoreType.DMA,
            pltpu.SemaphoreType.DMA,
        ],
    )
    def seg_kernel(y_hbm, src_hbm, dst_hbm, out_hbm,
                   srcb, dstb, rows0, rows1, acc, sem0, sem1):
        c = lax.axis_index("c")
        s = lax.axis_index("s")
        w = c * NS + s

        @pl.loop(0, K)
        def _(r):
            @pl.loop(0, d, step=LANES)
            def _(cc):
                rows0[pl.ds(r, 1), pl.ds(cc, LANES)] = (
                    jnp.zeros((1, LANES), jnp.float32))

        base_row = s * rpt
        nfull = rpt // K
        rem = rpt - nfull * K

        @pl.loop(0, nfull)
        def _(k):
            pltpu.sync_copy(rows0, acc.at[pl.ds(base_row + k * K, K)])

        if rem:
            pltpu.sync_copy(rows0.at[pl.ds(0, rem)],
                            acc.at[pl.ds(base_row + nfull * K, rem)])

        pltpu.sync_copy(src_hbm.at[pl.ds(w * nbt, nbt)], srcb)
        pltpu.sync_copy(dst_hbm.at[pl.ds(w * nbt, nbt)], dstb)
        plsc.subcore_barrier()

        def issue(j, rbuf, sem):
            pltpu.make_async_copy(y_hbm.at[srcb.at[j]], rbuf, sem).start()

        def wait(j, rbuf, sem):
            pltpu.make_async_copy(y_hbm.at[srcb.at[j]], rbuf, sem).wait()

        def scat(j, rbuf):
            pltpu.sync_copy(rbuf, acc.at[dstb.at[j]], add=True)

        issue(0, rows0, sem0)

        @pl.loop(0, nbt - 1, step=2)
        def _(j):
            issue(j + 1, rows1, sem1)
            wait(j, rows0, sem0)
            scat(j, rows0)
            issue(j + 2, rows0, sem0)
            wait(j + 1, rows1, sem1)
            scat(j + 1, rows1)

        wait(nbt - 1, rows0, sem0)
        scat(nbt - 1, rows0)

        plsc.subcore_barrier()
        pltpu.sync_copy(acc.at[pl.ds(base_row, rpt)],
                        out_hbm.at[c, pl.ds(base_row, rpt)])

    return seg_kernel(y, src2d, dst2d)


def _tc_scale_matmul(degp, x, w1):
    n, din = x.shape
    h = w1.shape[1]
    r = 2000

    def body(degp_ref, x_ref, w_ref, y_ref, dis_ref):
        dp = degp_ref[...]
        deg = dp[0, :, 0:1] + dp[1, :, 0:1] + 1.0
        dis = lax.rsqrt(deg)
        xw = jnp.dot(x_ref[...], w_ref[...],
                     preferred_element_type=jnp.float32)
        y_ref[...] = dis * xw
        dis_ref[...] = dis

    return pl.pallas_call(
        body,
        grid=(n // r,),
        in_specs=[
            pl.BlockSpec((NC, r, LANES), lambda i: (0, i, 0)),
            pl.BlockSpec((r, din), lambda i: (i, 0)),
            pl.BlockSpec((din, h), lambda i: (0, 0)),
        ],
        out_specs=[
            pl.BlockSpec((r, h), lambda i: (i, 0)),
            pl.BlockSpec((r, 1), lambda i: (i, 0)),
        ],
        out_shape=[
            jax.ShapeDtypeStruct((n, h), jnp.float32),
            jax.ShapeDtypeStruct((n, 1), jnp.float32),
        ],
    )(degp, x, w1)


def _tc_mid(acc1, y1, dis, b1, w2):
    n, hdim = y1.shape
    dout = w2.shape[1]
    r = 2000

    def body(acc_ref, y_ref, dis_ref, b_ref, w_ref, y2_ref):
        a = acc_ref[...]
        tot = a[0] + a[1] + y_ref[...]
        dis = dis_ref[...]
        hval = jnp.maximum(dis * tot + b_ref[...], 0.0)
        y2_ref[...] = dis * jnp.dot(hval, w_ref[...],
                                    preferred_element_type=jnp.float32)

    return pl.pallas_call(
        body,
        grid=(n // r,),
        in_specs=[
            pl.BlockSpec((NC, r, hdim), lambda i: (0, i, 0)),
            pl.BlockSpec((r, hdim), lambda i: (i, 0)),
            pl.BlockSpec((r, 1), lambda i: (i, 0)),
            pl.BlockSpec((1, hdim), lambda i: (0, 0)),
            pl.BlockSpec((hdim, dout), lambda i: (0, 0)),
        ],
        out_specs=pl.BlockSpec((r, dout), lambda i: (i, 0)),
        out_shape=jax.ShapeDtypeStruct((n, dout), jnp.float32),
    )(acc1, y1, dis, b1, w2)


def _tc_final(acc2, y2, dis, b2):
    n, dout = y2.shape
    r = 2000

    def body(acc_ref, y_ref, dis_ref, b_ref, o_ref):
        a = acc_ref[...]
        o_ref[...] = (dis_ref[...] * (a[0] + a[1] + y_ref[...])
                      + b_ref[...])

    return pl.pallas_call(
        body,
        grid=(n // r,),
        in_specs=[
            pl.BlockSpec((NC, r, dout), lambda i: (0, i, 0)),
            pl.BlockSpec((r, dout), lambda i: (i, 0)),
            pl.BlockSpec((r, 1), lambda i: (i, 0)),
            pl.BlockSpec((1, dout), lambda i: (0, 0)),
        ],
        out_specs=pl.BlockSpec((r, dout), lambda i: (i, 0)),
        out_shape=jax.ShapeDtypeStruct((n, dout), jnp.float32),
    )(acc2, y2, dis, b2)


def kernel(x, edge, W1, b1, W2, b2):
    n, _ = x.shape
    e = edge.shape[1]
    assert e % (K * NW) == 0 and n % NS == 0

    edge_i32 = edge.astype(jnp.int32)
    src2d = edge_i32[0].reshape(e // K, K)
    dst2d = edge_i32[1].reshape(e // K, K)

    degp = _sc_degree(dst2d, n)
    y1, dis = _tc_scale_matmul(degp, x, W1)
    acc1 = _sc_segment_sum(y1, src2d, dst2d)
    y2 = _tc_mid(acc1, y1, dis, b1.reshape(1, -1), W2)
    acc2 = _sc_segment_sum(y2, src2d, dst2d)
    return _tc_final(acc2, y2, dis, b2.reshape(1, -1))


# speedup vs baseline: 25.5156x; 25.5156x over previous
import functools

import jax
import jax.numpy as jnp
from jax import lax
from jax.experimental import pallas as pl
from jax.experimental.pallas import tpu as pltpu
from jax.experimental.pallas import tpu_sc as plsc

NC = 2
NS = 16
NW = NC * NS
K = 80
LANES = 16
NPAD = 10240


def _sc_degree(dst3d):
    nbt = dst3d.shape[1]
    rpt = NPAD // NS
    mesh = plsc.VectorSubcoreMesh(core_axis_name="c", subcore_axis_name="s")

    @functools.partial(
        pl.kernel,
        mesh=mesh,
        compiler_params=pltpu.CompilerParams(use_tc_tiling_on_sc=False),
        out_type=jax.ShapeDtypeStruct((NC, NPAD, LANES), jnp.float32),
        scratch_types=[
            pltpu.VMEM((nbt, K), jnp.int32),
            pltpu.VMEM((K, LANES), jnp.float32),
            pltpu.VMEM((K, LANES), jnp.float32),
            pltpu.VMEM_SHARED((NPAD, LANES), jnp.float32),
        ],
    )
    def deg_kernel(dst_hbm, out_hbm, dstb, ones_v, zero_v, acc):
        c = lax.axis_index("c")
        s = lax.axis_index("s")
        w = c * NS + s

        @pl.loop(0, K)
        def _(r):
            ones_v[pl.ds(r, 1), :] = jnp.ones((1, LANES), jnp.float32)
            zero_v[pl.ds(r, 1), :] = jnp.zeros((1, LANES), jnp.float32)

        base_row = s * rpt
        nfull = rpt // K

        @pl.loop(0, nfull)
        def _(k):
            pltpu.sync_copy(zero_v, acc.at[pl.ds(base_row + k * K, K)])

        pltpu.sync_copy(dst_hbm.at[w], dstb)
        plsc.subcore_barrier()

        @pl.loop(0, nbt)
        def _(j):
            pltpu.sync_copy(ones_v, acc.at[dstb.at[j]], add=True)

        plsc.subcore_barrier()
        pltpu.sync_copy(acc.at[pl.ds(base_row, rpt)],
                        out_hbm.at[c, pl.ds(base_row, rpt)])

    return deg_kernel(dst3d)


def _sc_segment_sum(ysplit, src3d, dst3d):
    dh = ysplit.shape[2]
    nbt = src3d.shape[1]
    rpt = NPAD // NS
    assert nbt % 2 == 0
    mesh = plsc.VectorSubcoreMesh(core_axis_name="c", subcore_axis_name="s")

    @functools.partial(
        pl.kernel,
        mesh=mesh,
        compiler_params=pltpu.CompilerParams(use_tc_tiling_on_sc=False),
        out_type=jax.ShapeDtypeStruct((NC, NPAD, dh), jnp.float32),
        scratch_types=[
            pltpu.VMEM((nbt, K), jnp.int32),
            pltpu.VMEM((nbt, K), jnp.int32),
            pltpu.VMEM((K, dh), jnp.float32),
            pltpu.VMEM((K, dh), jnp.float32),
            pltpu.VMEM_SHARED((NPAD, dh), jnp.float32),
            pltpu.SemaphoreType.DMA,
            pltpu.SemaphoreType.DMA,
        ],
    )
    def seg_kernel(y_hbm, src_hbm, dst_hbm, out_hbm,
                   srcb, dstb, rows0, rows1, acc, sem0, sem1):
        c = lax.axis_index("c")
        s = lax.axis_index("s")

        @pl.loop(0, K)
        def _(r):
            @pl.loop(0, dh, step=LANES)
            def _(cc):
                rows0[pl.ds(r, 1), pl.ds(cc, LANES)] = (
                    jnp.zeros((1, LANES), jnp.float32))

        base_row = s * rpt
        nfull = rpt // K

        @pl.loop(0, nfull)
        def _(k):
            pltpu.sync_copy(rows0, acc.at[pl.ds(base_row + k * K, K)])

        pltpu.sync_copy(src_hbm.at[s], srcb)
        pltpu.sync_copy(dst_hbm.at[s], dstb)
        plsc.subcore_barrier()

        yc = y_hbm.at[c]

        def issue(j, rbuf, sem):
            pltpu.make_async_copy(yc.at[srcb.at[j]], rbuf, sem).start()

        def wait(j, rbuf, sem):
            pltpu.make_async_copy(yc.at[srcb.at[j]], rbuf, sem).wait()

        def scat(j, rbuf):
            pltpu.sync_copy(rbuf, acc.at[dstb.at[j]], add=True)

        issue(0, rows0, sem0)

        @pl.loop(0, nbt, step=2)
        def _(j):
            issue(j + 1, rows1, sem1)
            wait(j, rows0, sem0)
            scat(j, rows0)

            @pl.when(j + 2 < nbt)
            def _():
                issue(j + 2, rows0, sem0)

            wait(j + 1, rows1, sem1)
            scat(j + 1, rows1)

        plsc.subcore_barrier()
        pltpu.sync_copy(acc.at[pl.ds(base_row, rpt)],
                        out_hbm.at[c, pl.ds(base_row, rpt)])

    return seg_kernel(ysplit, src3d, dst3d)


def _tc_scale_matmul(degp, x, w1):
    n, din = x.shape
    h = w1.shape[1]
    dh = h // NC
    r = 2000

    def body(degp_ref, x_ref, w_ref, y_ref, dis_ref):
        dp = degp_ref[...]
        deg = dp[0, :, 0:1] + dp[1, :, 0:1] + 1.0
        dis = lax.rsqrt(deg)
        xw = jnp.dot(x_ref[...], w_ref[...],
                     preferred_element_type=jnp.float32)
        yv = dis * xw
        y_ref[0] = yv[:, :dh]
        y_ref[1] = yv[:, dh:]
        dis_ref[...] = dis

    return pl.pallas_call(
        body,
        grid=(n // r,),
        in_specs=[
            pl.BlockSpec((NC, r, LANES), lambda i: (0, i, 0)),
            pl.BlockSpec((r, din), lambda i: (i, 0)),
            pl.BlockSpec((din, h), lambda i: (0, 0)),
        ],
        out_specs=[
            pl.BlockSpec((NC, r, dh), lambda i: (0, i, 0)),
            pl.BlockSpec((r, 1), lambda i: (i, 0)),
        ],
        out_shape=[
            jax.ShapeDtypeStruct((NC, n, dh), jnp.float32),
            jax.ShapeDtypeStruct((n, 1), jnp.float32),
        ],
    )(degp, x, w1)


def _tc_mid(acc1, y1, dis, b1, w2):
    n = dis.shape[0]
    dh1 = y1.shape[2]
    hdim = NC * dh1
    dout = w2.shape[1]
    dh2 = dout // NC
    r = 2000

    def body(acc_ref, y_ref, dis_ref, b_ref, w_ref, y2_ref):
        a = acc_ref[...]
        yv = y_ref[...]
        tot = jnp.concatenate([a[0] + yv[0], a[1] + yv[1]], axis=1)
        dis = dis_ref[...]
        hval = jnp.maximum(dis * tot + b_ref[...], 0.0)
        y2v = dis * jnp.dot(hval, w_ref[...],
                            preferred_element_type=jnp.float32)
        y2_ref[0] = y2v[:, :dh2]
        y2_ref[1] = y2v[:, dh2:]

    return pl.pallas_call(
        body,
        grid=(n // r,),
        in_specs=[
            pl.BlockSpec((NC, r, dh1), lambda i: (0, i, 0)),
            pl.BlockSpec((NC, r, dh1), lambda i: (0, i, 0)),
            pl.BlockSpec((r, 1), lambda i: (i, 0)),
            pl.BlockSpec((1, hdim), lambda i: (0, 0)),
            pl.BlockSpec((hdim, dout), lambda i: (0, 0)),
        ],
        out_specs=pl.BlockSpec((NC, r, dh2), lambda i: (0, i, 0)),
        out_shape=jax.ShapeDtypeStruct((NC, n, dh2), jnp.float32),
    )(acc1, y1, dis, b1, w2)


def _tc_final(acc2, y2, dis, b2):
    n = dis.shape[0]
    dh2 = y2.shape[2]
    dout = NC * dh2
    r = 2000

    def body(acc_ref, y_ref, dis_ref, b_ref, o_ref):
        a = acc_ref[...]
        yv = y_ref[...]
        tot = jnp.concatenate([a[0] + yv[0], a[1] + yv[1]], axis=1)
        o_ref[...] = dis_ref[...] * tot + b_ref[...]

    return pl.pallas_call(
        body,
        grid=(n // r,),
        in_specs=[
            pl.BlockSpec((NC, r, dh2), lambda i: (0, i, 0)),
            pl.BlockSpec((NC, r, dh2), lambda i: (0, i, 0)),
            pl.BlockSpec((r, 1), lambda i: (i, 0)),
            pl.BlockSpec((1, dout), lambda i: (0, 0)),
        ],
        out_specs=pl.BlockSpec((r, dout), lambda i: (i, 0)),
        out_shape=jax.ShapeDtypeStruct((n, dout), jnp.float32),
    )(acc2, y2, dis, b2)


def kernel(x, edge, W1, b1, W2, b2):
    n, _ = x.shape
    e = edge.shape[1]
    assert e % (K * NW) == 0 and n % NS == 0

    edge_i32 = edge.astype(jnp.int32)
    nbt_deg = e // (K * NW)
    src_seg = edge_i32[0].reshape(NS, e // (K * NS), K)
    dst_seg = edge_i32[1].reshape(NS, e // (K * NS), K)
    dst_deg = edge_i32[1].reshape(NW, nbt_deg, K)

    degp = _sc_degree(dst_deg)
    y1, dis = _tc_scale_matmul(degp, x, W1)
    acc1 = _sc_segment_sum(y1, src_seg, dst_seg)
    y2 = _tc_mid(acc1, y1, dis, b1.reshape(1, -1), W2)
    acc2 = _sc_segment_sum(y2, src_seg, dst_seg)
    return _tc_final(acc2, y2, dis, b2.reshape(1, -1))
